# baseline (speedup 1.0000x reference)
import jax
import jax.numpy as jnp
from jax.experimental import pallas as pl

_B = 256
_MAXN = 24
_NUM_TYPES = 10
_MAXPOS = 9
_HID = 301
_LAT = 56


def _mm_t(x, w):
    return jax.lax.dot_general(x, w, (((1,), (1,)), ((), ())),
                               preferred_element_type=jnp.float32)


def _kern(a_ref, types_ref, pos_ref, rcg_ref,
          wb_ref, wt_ref, wp_ref, ws_ref, out_ref):
    f32 = jnp.float32
    H = _HID
    whh_r = wb_ref[0:H, :]
    whh_z = wb_ref[304:304 + H, :]
    whh_n = wb_ref[608:608 + H, :]
    wg_h = wb_ref[912:912 + H, :]
    wm_h = wb_ref[1216:1216 + H, :]
    wfc_h = wb_ref[1520:1520 + 2 * _LAT, :]
    bih_r = wb_ref[1632:1633, :]
    bih_z = wb_ref[1633:1634, :]
    bih_n = wb_ref[1634:1635, :]
    bhh_r = wb_ref[1635:1636, :]
    bhh_z = wb_ref[1636:1637, :]
    bhh_n = wb_ref[1637:1638, :]
    bg = wb_ref[1640:1641, :]
    wih_t_r = wt_ref[0:H, :]
    wih_t_z = wt_ref[304:304 + H, :]
    wih_t_n = wt_ref[608:608 + H, :]
    wih_p_r = wp_ref[0:H, :]
    wih_p_z = wp_ref[304:304 + H, :]
    wih_p_n = wp_ref[608:608 + H, :]
    wg_p = wp_ref[912:912 + H, :]
    wm_p = wp_ref[1216:1216 + H, :]
    wdf1 = ws_ref[0:16, 0:27]
    wdf2 = ws_ref[16:24, 0:16]
    wfc_f = ws_ref[24:136, 0:8]
    bdf1 = ws_ref[136:137, 0:16]
    bdf2 = ws_ref[144:145, 0:8]
    bfc = ws_ref[152:153, 0:2 * _LAT]

    types = types_ref[...]
    posq = pos_ref[...]
    iota_t = jax.lax.broadcasted_iota(jnp.int32, (_B, _NUM_TYPES), 1)
    iota_p = jax.lax.broadcasted_iota(jnp.int32, (_B, _MAXPOS), 1)

    grows = []
    hv = None
    for v in range(_MAXN):
        if v == 0:
            hin = jnp.zeros((_B, _HID), f32)
        else:
            terms = [jnp.where(a_ref[:, 24 * v + u:24 * v + u + 1] < 0.3,
                               grows[u], 0.0)
                     for u in range(v)]
            while len(terms) > 1:
                terms = [terms[i] + terms[i + 1] if i + 1 < len(terms)
                         else terms[i] for i in range(0, len(terms), 2)]
            hin = terms[0]
        oh_t = jnp.where(types[:, v:v + 1] == iota_t, 1.0, 0.0)
        oh_p = jnp.where(posq[:, v:v + 1] == iota_p, 1.0, 0.0)
        r = jax.nn.sigmoid(_mm_t(oh_t, wih_t_r) + _mm_t(oh_p, wih_p_r)
                           + bih_r + _mm_t(hin, whh_r) + bhh_r)
        z = jax.nn.sigmoid(_mm_t(oh_t, wih_t_z) + _mm_t(oh_p, wih_p_z)
                           + bih_z + _mm_t(hin, whh_z) + bhh_z)
        n = jnp.tanh(_mm_t(oh_t, wih_t_n) + _mm_t(oh_p, wih_p_n)
                     + bih_n + r * (_mm_t(hin, whh_n) + bhh_n))
        hv = (1.0 - z) * n + z * hin
        if v < _MAXN - 1:
            gate = jax.nn.sigmoid(_mm_t(hv, wg_h) + _mm_t(oh_p, wg_p) + bg)
            grows.append(gate * (_mm_t(hv, wm_h) + _mm_t(oh_p, wm_p)))
    hg = hv

    j3 = jax.lax.broadcasted_iota(jnp.int32, (_B, _MAXN, 3 * _MAXPOS), 2)
    pj = j3 // 3
    kj = j3 - pj * 3
    niota = jax.lax.broadcasted_iota(jnp.int32, (_B, _MAXN, 3 * _MAXPOS), 1) + 1
    m27i = jnp.where(posq[:, :, None] == pj, niota, 0)
    nmax = jnp.max(m27i, axis=1)
    last = jnp.where((m27i == nmax[:, None, :]) & (m27i > 0), 1.0, 0.0)
    r3 = rcg_ref[...]
    rcg27 = (jnp.where(kj == 0, r3[:, :, 0:1], 0.0)
             + jnp.where(kj == 1, r3[:, :, 1:2], 0.0)
             + jnp.where(kj == 2, r3[:, :, 2:3], 0.0))
    df = jnp.sum(last * rcg27, axis=1)

    hdf = jnp.maximum(_mm_t(df, wdf1) + bdf1, 0.0)
    hdf = _mm_t(hdf, wdf2) + bdf2

    out_ref[...] = _mm_t(hg, wfc_h) + _mm_t(0.01 * hdf, wfc_f) + bfc


def kernel(node_types, node_pos, adj_rand, node_rcg, Wih, Whh, bih, bhh,
           Wg, bg, Wm, Wdf1, bdf1, Wdf2, bdf2, Wfc1, bfc1, Wfc2, bfc2):
    f32 = jnp.float32
    H = _HID
    NT = _NUM_TYPES
    a = adj_rand.transpose(0, 2, 1).reshape(_B, _MAXN * _MAXN)

    def _pad8(m):
        r = (-m.shape[0]) % 8
        return m if r == 0 else jnp.concatenate(
            [m, jnp.zeros((r, m.shape[1]), f32)], axis=0)

    wb = jnp.concatenate([
        _pad8(Whh[0:H]), _pad8(Whh[H:2 * H]), _pad8(Whh[2 * H:]),
        _pad8(Wg[:, :H]), _pad8(Wm[:, :H]),
        jnp.concatenate([Wfc1[:, :H], Wfc2[:, :H]], axis=0),
        bih.reshape(3, H), bhh.reshape(3, H),
        jnp.zeros((2, H), f32), bg[None, :], jnp.zeros((7, H), f32),
    ], axis=0)
    wt = jnp.concatenate([
        _pad8(Wih[0:H, :NT]), _pad8(Wih[H:2 * H, :NT]),
        _pad8(Wih[2 * H:, :NT]),
    ], axis=0)
    wp = jnp.concatenate([
        _pad8(Wih[0:H, NT:]), _pad8(Wih[H:2 * H, NT:]),
        _pad8(Wih[2 * H:, NT:]),
        _pad8(Wg[:, H:]), _pad8(Wm[:, H:]),
    ], axis=0)

    def _place(m, rows, cols):
        return jnp.pad(m, ((0, rows - m.shape[0]), (0, cols - m.shape[1])))

    ws = jnp.concatenate([
        _place(Wdf1, 16, 112),
        _place(Wdf2, 8, 112),
        _place(jnp.concatenate([Wfc1[:, H:], Wfc2[:, H:]], axis=0),
               112, 112),
        _place(bdf1[None, :], 8, 112),
        _place(bdf2[None, :], 8, 112),
        _place(jnp.concatenate([bfc1, bfc2])[None, :], 8, 112),
    ], axis=0)

    args = (a, node_types.astype(jnp.int32), node_pos.astype(jnp.int32),
            node_rcg, wb, wt, wp, ws)
    return pl.pallas_call(
        _kern,
        out_shape=jax.ShapeDtypeStruct((_B, 2 * _LAT), f32),
    )(*args)


# speedup vs baseline: 1.5670x; 1.5670x over previous
import jax
import jax.numpy as jnp
from jax.experimental import pallas as pl

_B = 256
_MAXN = 24
_NUM_TYPES = 10
_MAXPOS = 9
_HID = 301
_LAT = 56


def _mm(w, x):
    return jax.lax.dot_general(w, x, (((1,), (0,)), ((), ())),
                               preferred_element_type=jnp.float32)


def _mm_tn(xt, w):
    return jax.lax.dot_general(xt, w, (((0,), (1,)), ((), ())),
                               preferred_element_type=jnp.float32)


def _kern(a_ref, types_ref, pos_ref, rcg_ref,
          wih_t_r_ref, wih_t_z_ref, wih_t_n_ref,
          wih_p_r_ref, wih_p_z_ref, wih_p_n_ref,
          whh_r_ref, whh_z_ref, whh_n_ref,
          b6_ref,
          wg_h_ref, wg_p_ref, bg_ref,
          wm_h_ref, wm_p_ref,
          wdf1_ref, bdf1_ref, wdf2_ref, bdf2_ref,
          wfc_h_ref, wfc_f_ref, bfc_ref,
          out_ref):
    f32 = jnp.float32
    wih_t_r = wih_t_r_ref[...]
    wih_t_z = wih_t_z_ref[...]
    wih_t_n = wih_t_n_ref[...]
    wih_p_r = wih_p_r_ref[...]
    wih_p_z = wih_p_z_ref[...]
    wih_p_n = wih_p_n_ref[...]
    whh_r = whh_r_ref[...]
    whh_z = whh_z_ref[...]
    whh_n = whh_n_ref[...]
    b6 = b6_ref[...]
    bih_r, bih_z, bih_n = b6[:, 0:1], b6[:, 1:2], b6[:, 2:3]
    bhh_r, bhh_z, bhh_n = b6[:, 3:4], b6[:, 4:5], b6[:, 5:6]
    wg_h = wg_h_ref[...]
    wg_p = wg_p_ref[...]
    bg = bg_ref[...]
    wm_h = wm_h_ref[...]
    wm_p = wm_p_ref[...]

    types = types_ref[...]
    posq = pos_ref[...]
    iota_t = jax.lax.broadcasted_iota(jnp.int32, (_NUM_TYPES, _B), 0)
    iota_p = jax.lax.broadcasted_iota(jnp.int32, (_MAXPOS, _B), 0)

    grows = []
    hv = None
    for v in range(_MAXN):
        if v == 0:
            hin = jnp.zeros((_HID, _B), f32)
        else:
            terms = [jnp.where(a_ref[24 * v + u:24 * v + u + 1, :] < 0.3,
                               grows[u], 0.0)
                     for u in range(v)]
            while len(terms) > 1:
                terms = [terms[i] + terms[i + 1] if i + 1 < len(terms)
                         else terms[i] for i in range(0, len(terms), 2)]
            hin = terms[0]
        oh_t = jnp.where(types[v:v + 1, :] == iota_t, 1.0, 0.0)
        oh_p = jnp.where(posq[v:v + 1, :] == iota_p, 1.0, 0.0)
        r = jax.nn.sigmoid(_mm(wih_t_r, oh_t) + _mm(wih_p_r, oh_p)
                           + bih_r + _mm(whh_r, hin) + bhh_r)
        z = jax.nn.sigmoid(_mm(wih_t_z, oh_t) + _mm(wih_p_z, oh_p)
                           + bih_z + _mm(whh_z, hin) + bhh_z)
        n = jnp.tanh(_mm(wih_t_n, oh_t) + _mm(wih_p_n, oh_p)
                     + bih_n + r * (_mm(whh_n, hin) + bhh_n))
        hv = (1.0 - z) * n + z * hin
        if v < _MAXN - 1:
            gate = jax.nn.sigmoid(_mm(wg_h, hv) + _mm(wg_p, oh_p) + bg)
            grows.append(gate * (_mm(wm_h, hv) + _mm(wm_p, oh_p)))
    hg = hv

    j3 = jax.lax.broadcasted_iota(jnp.int32, (_MAXN, 3 * _MAXPOS, _B), 1)
    pj = j3 // 3
    kj = j3 - pj * 3
    niota = jax.lax.broadcasted_iota(jnp.int32, (_MAXN, 3 * _MAXPOS, _B), 0) + 1
    m27i = jnp.where(posq[:, None, :] == pj, niota, 0)
    nmax = jnp.max(m27i, axis=0)
    last = jnp.where((m27i == nmax[None, :, :]) & (m27i > 0), 1.0, 0.0)
    r3 = rcg_ref[...]
    rcg27 = (jnp.where(kj == 0, r3[:, 0:1, :], 0.0)
             + jnp.where(kj == 1, r3[:, 1:2, :], 0.0)
             + jnp.where(kj == 2, r3[:, 2:3, :], 0.0))
    df = jnp.sum(last * rcg27, axis=0)

    hdf = jnp.maximum(_mm(wdf1_ref[...], df) + bdf1_ref[...], 0.0)
    hdf = _mm(wdf2_ref[...], hdf) + bdf2_ref[...]

    out_ref[...] = (_mm_tn(hg, wfc_h_ref[...])
                    + _mm_tn(0.01 * hdf, wfc_f_ref[...]) + bfc_ref[...])


def kernel(node_types, node_pos, adj_rand, node_rcg, Wih, Whh, bih, bhh,
           Wg, bg, Wm, Wdf1, bdf1, Wdf2, bdf2, Wfc1, bfc1, Wfc2, bfc2):
    H = _HID
    NT = _NUM_TYPES
    a = adj_rand.transpose(2, 1, 0).reshape(_MAXN * _MAXN, _B)

    args = (
        a, node_types.T.astype(jnp.int32), node_pos.T.astype(jnp.int32),
        node_rcg.transpose(1, 2, 0),
        Wih[0:H, :NT], Wih[H:2 * H, :NT], Wih[2 * H:, :NT],
        Wih[0:H, NT:], Wih[H:2 * H, NT:], Wih[2 * H:, NT:],
        Whh[0:H], Whh[H:2 * H], Whh[2 * H:],
        jnp.stack([bih[0:H], bih[H:2 * H], bih[2 * H:],
                   bhh[0:H], bhh[H:2 * H], bhh[2 * H:]], axis=1),
        Wg[:, :H], Wg[:, H:], bg[:, None],
        Wm[:, :H], Wm[:, H:],
        Wdf1, bdf1[:, None], Wdf2, bdf2[:, None],
        jnp.concatenate([Wfc1[:, :H], Wfc2[:, :H]], axis=0),
        jnp.concatenate([Wfc1[:, H:], Wfc2[:, H:]], axis=0),
        jnp.concatenate([bfc1, bfc2])[None, :],
    )
    return pl.pallas_call(
        _kern,
        out_shape=jax.ShapeDtypeStruct((_B, 2 * _LAT), jnp.float32),
    )(*args)
